# baseline (speedup 1.0000x reference)
import jax
import jax.numpy as jnp
from jax import lax
from jax.experimental import pallas as pl
from jax.experimental.pallas import tpu as pltpu
from jax.experimental.pallas import tpu_sc as plsc

N_NODES = 50000
N_INC = 800000
SEQ = 16
FIN = 6
H = 32
NCORE = 2
NSUB = 16
NTILE = NCORE * NSUB
PER_TILE = N_INC // NTILE
CHUNK = 125
NCHUNK = PER_TILE // CHUNK
IDX_BLK = 40
NREFILL = NCHUNK // IDX_BLK
IDX_ROWS = N_INC // CHUNK
N_PAD = 50048
ROWS_PER_TILE = N_PAD // NSUB
CW = 8
BLK = 2000
GRID = N_NODES // BLK

_SC_MESH = dict(core_axis_name="c", subcore_axis_name="s",
                num_cores=NCORE, num_subcores=NSUB)


def _sc_pass_body(table, idxg, idxs, zrows, out, acc,
                  igv, isv, rows0, rows1, sem0, sem1):
    c = lax.axis_index("c")
    s = lax.axis_index("s")
    w = c * NSUB + s
    sl = pl.ds(s * ROWS_PER_TILE, ROWS_PER_TILE)
    pltpu.sync_copy(zrows, acc.at[sl])
    plsc.subcore_barrier()

    def outer(j, carry):
        base = w * NCHUNK + j * IDX_BLK
        pltpu.sync_copy(idxg.at[pl.ds(base, IDX_BLK)], igv)
        pltpu.sync_copy(idxs.at[pl.ds(base, IDX_BLK)], isv)
        pltpu.async_copy(table.at[igv.at[0]], rows0, sem0)

        def body(k, carry2):
            i0 = 2 * k
            pltpu.async_copy(table.at[igv.at[i0 + 1]], rows1, sem1)
            pltpu.make_async_copy(table.at[igv.at[i0]], rows0, sem0).wait()
            pltpu.sync_copy(rows0, acc.at[isv.at[i0]], add=True)

            @pl.when(k < IDX_BLK // 2 - 1)
            def _():
                pltpu.async_copy(table.at[igv.at[i0 + 2]], rows0, sem0)

            pltpu.make_async_copy(table.at[igv.at[i0 + 1]], rows1, sem1).wait()
            pltpu.sync_copy(rows1, acc.at[isv.at[i0 + 1]], add=True)
            return carry2

        lax.fori_loop(0, IDX_BLK // 2, body, 0)
        return carry

    lax.fori_loop(0, NREFILL, outer, 0)
    plsc.subcore_barrier()

    @pl.when(c == 0)
    def _():
        pltpu.sync_copy(acc.at[sl], out.at[0, sl])

    @pl.when(c == 1)
    def _():
        pltpu.sync_copy(acc.at[sl], out.at[1, sl])


def _sc_pass(table, idxg, idxs, zrows):
    return pl.kernel(
        _sc_pass_body,
        out_type=jax.ShapeDtypeStruct((NCORE, N_PAD, H), jnp.float32),
        mesh=plsc.VectorSubcoreMesh(**_SC_MESH),
        scratch_types=[
            pltpu.VMEM_SHARED((N_PAD, H), jnp.float32),
            pltpu.VMEM((IDX_BLK, CHUNK), jnp.int32),
            pltpu.VMEM((IDX_BLK, CHUNK), jnp.int32),
            pltpu.VMEM((CHUNK, H), jnp.float32),
            pltpu.VMEM((CHUNK, H), jnp.float32),
            pltpu.SemaphoreType.DMA,
            pltpu.SemaphoreType.DMA,
        ],
        compiler_params=pltpu.CompilerParams(use_tc_tiling_on_sc=False),
    )(table, idxg, idxs, zrows)


def _sc_counts_body(idxg, idxs, zrows, ones, outd, outb, accd, accb,
                    igv, isv, onev):
    c = lax.axis_index("c")
    s = lax.axis_index("s")
    w = c * NSUB + s
    sl = pl.ds(s * ROWS_PER_TILE, ROWS_PER_TILE)
    pltpu.sync_copy(zrows, accd.at[sl])
    pltpu.sync_copy(zrows, accb.at[sl])
    pltpu.sync_copy(ones, onev)
    plsc.subcore_barrier()

    def outer(j, carry):
        base = w * NCHUNK + j * IDX_BLK
        pltpu.sync_copy(idxg.at[pl.ds(base, IDX_BLK)], igv)
        pltpu.sync_copy(idxs.at[pl.ds(base, IDX_BLK)], isv)

        def body(i, carry2):
            pltpu.sync_copy(onev, accd.at[igv.at[i]], add=True)
            pltpu.sync_copy(onev, accb.at[isv.at[i]], add=True)
            return carry2

        lax.fori_loop(0, IDX_BLK, body, 0)
        return carry

    lax.fori_loop(0, NREFILL, outer, 0)
    plsc.subcore_barrier()

    @pl.when(c == 0)
    def _():
        pltpu.sync_copy(accd.at[sl], outd.at[0, sl])
        pltpu.sync_copy(accb.at[sl], outb.at[0, sl])

    @pl.when(c == 1)
    def _():
        pltpu.sync_copy(accd.at[sl], outd.at[1, sl])
        pltpu.sync_copy(accb.at[sl], outb.at[1, sl])


def _sc_counts(idxg, idxs):
    zrows = jnp.zeros((ROWS_PER_TILE, CW), jnp.float32)
    ones = jnp.ones((CHUNK, CW), jnp.float32)
    return pl.kernel(
        _sc_counts_body,
        out_type=(jax.ShapeDtypeStruct((NCORE, N_PAD, CW), jnp.float32),
                  jax.ShapeDtypeStruct((NCORE, N_PAD, CW), jnp.float32)),
        mesh=plsc.VectorSubcoreMesh(**_SC_MESH),
        scratch_types=[
            pltpu.VMEM_SHARED((N_PAD, CW), jnp.float32),
            pltpu.VMEM_SHARED((N_PAD, CW), jnp.float32),
            pltpu.VMEM((IDX_BLK, CHUNK), jnp.int32),
            pltpu.VMEM((IDX_BLK, CHUNK), jnp.int32),
            pltpu.VMEM((CHUNK, CW), jnp.float32),
        ],
        compiler_params=pltpu.CompilerParams(use_tc_tiling_on_sc=False),
    )(idxg, idxs, zrows, ones)


PACK = 4
NROWS = N_NODES // PACK
NROWS_PAD = 12800
GROWS = 3200
GGRID = NROWS_PAD // GROWS
XW = PACK * SEQ * FIN
HW = PACK * H


def _sigm(v):
    return 0.5 * (jnp.tanh(0.5 * v) + 1.0)


def _gru_body(x_ref, wxr_ref, wxz_ref, wxn_ref, whr_ref, whz_ref, whn_ref,
              br_ref, bz_ref, bni_ref, bnh_ref, w1_ref, out_ref):
    x = x_ref[...]
    whr = whr_ref[...]
    whz = whz_ref[...]
    whn = whn_ref[...]
    br = br_ref[...]
    bz = bz_ref[...]
    bni = bni_ref[...]
    bnh = bnh_ref[...]
    h = jnp.zeros((GROWS, HW), jnp.float32)
    for t in range(SEQ):
        gr = (jnp.dot(x, wxr_ref[t], preferred_element_type=jnp.float32)
              + jnp.dot(h, whr, preferred_element_type=jnp.float32) + br)
        gz = (jnp.dot(x, wxz_ref[t], preferred_element_type=jnp.float32)
              + jnp.dot(h, whz, preferred_element_type=jnp.float32) + bz)
        r = _sigm(gr)
        z = _sigm(gz)
        hn = jnp.dot(h, whn, preferred_element_type=jnp.float32) + bnh
        gn = jnp.dot(x, wxn_ref[t], preferred_element_type=jnp.float32) + bni
        n = jnp.tanh(gn + r * hn)
        h = (1.0 - z) * n + z * h
    o = jnp.where(h > 0, h, 0.01 * h)
    out_ref[...] = jnp.dot(o, w1_ref[...], preferred_element_type=jnp.float32)


def _tc_gru(x4, wx, wh, bg, w14):
    full = lambda shape: pl.BlockSpec(shape, lambda i: tuple(0 for _ in shape))
    return pl.pallas_call(
        _gru_body,
        grid=(GGRID,),
        in_specs=[
            pl.BlockSpec((GROWS, XW), lambda i: (i, 0)),
            full((SEQ, XW, HW)),
            full((SEQ, XW, HW)),
            full((SEQ, XW, HW)),
            full((HW, HW)),
            full((HW, HW)),
            full((HW, HW)),
            full((1, HW)),
            full((1, HW)),
            full((1, HW)),
            full((1, HW)),
            full((HW, HW)),
        ],
        out_specs=pl.BlockSpec((GROWS, HW), lambda i: (i, 0)),
        out_shape=jax.ShapeDtypeStruct((NROWS_PAD, HW), jnp.float32),
    )(x4, wx[0], wx[1], wx[2], wh[0], wh[1], wh[2],
      bg[0], bg[1], bg[2], bg[3], w14)


def _inv_body(cd_ref, cb_ref, dinv_ref, binv_ref):
    cd = cd_ref[0, :, :1] + cd_ref[1, :, :1]
    cb = cb_ref[0, :, :1] + cb_ref[1, :, :1]
    dinv_ref[...] = jnp.where(cd > 0, 1.0 / cd, 0.0)
    binv_ref[...] = jnp.where(cb > 0, 1.0 / cb, 0.0)


def _tc_inv(cd, cb):
    return pl.pallas_call(
        _inv_body,
        grid=(GRID,),
        in_specs=[
            pl.BlockSpec((NCORE, BLK, CW), lambda i: (0, i, 0)),
            pl.BlockSpec((NCORE, BLK, CW), lambda i: (0, i, 0)),
        ],
        out_specs=[
            pl.BlockSpec((BLK, 1), lambda i: (i, 0)),
            pl.BlockSpec((BLK, 1), lambda i: (i, 0)),
        ],
        out_shape=[jax.ShapeDtypeStruct((N_NODES, 1), jnp.float32),
                   jax.ShapeDtypeStruct((N_NODES, 1), jnp.float32)],
    )(cd, cb)


def _edge_flush_body(p_ref, binv_ref, out_ref):
    out_ref[...] = (p_ref[0] + p_ref[1]) * binv_ref[...]


def _tc_edge_flush(p, binv):
    return pl.pallas_call(
        _edge_flush_body,
        grid=(GRID,),
        in_specs=[
            pl.BlockSpec((NCORE, BLK, H), lambda i: (0, i, 0)),
            pl.BlockSpec((BLK, 1), lambda i: (i, 0)),
        ],
        out_specs=pl.BlockSpec((BLK, H), lambda i: (i, 0)),
        out_shape=jax.ShapeDtypeStruct((N_NODES, H), jnp.float32),
    )(p, binv)


def _node_flush_body(p_ref, dinv_ref, b_ref, w_ref, out_ref):
    y = (p_ref[0] + p_ref[1]) * dinv_ref[...] + b_ref[...]
    y = jnp.where(y > 0, y, 0.2 * y)
    out_ref[...] = jnp.dot(y, w_ref[...], preferred_element_type=jnp.float32)


def _tc_node_flush(p, dinv, b2d, w_t):
    return pl.pallas_call(
        _node_flush_body,
        grid=(GRID,),
        in_specs=[
            pl.BlockSpec((NCORE, BLK, H), lambda i: (0, i, 0)),
            pl.BlockSpec((BLK, 1), lambda i: (i, 0)),
            pl.BlockSpec((1, H), lambda i: (0, 0)),
            pl.BlockSpec((H, H), lambda i: (0, 0)),
        ],
        out_specs=pl.BlockSpec((BLK, H), lambda i: (i, 0)),
        out_shape=jax.ShapeDtypeStruct((N_NODES, H), jnp.float32),
    )(p, dinv, b2d, w_t)


def _final_body(p_ref, dinv_ref, b_ref, wl_ref, bl_ref, out_ref):
    y = (p_ref[0] + p_ref[1]) * dinv_ref[...] + b_ref[...]
    y = jnp.where(y > 0, y, 0.2 * y)
    o = jnp.dot(y, wl_ref[...], preferred_element_type=jnp.float32) + bl_ref[...]
    out_ref[...] = jnp.where(o > 0, o, 0.01 * o)


def _tc_final(p, dinv, b2d, wl_t, bl2d):
    n_out = wl_t.shape[1]
    return pl.pallas_call(
        _final_body,
        grid=(GRID,),
        in_specs=[
            pl.BlockSpec((NCORE, BLK, H), lambda i: (0, i, 0)),
            pl.BlockSpec((BLK, 1), lambda i: (i, 0)),
            pl.BlockSpec((1, H), lambda i: (0, 0)),
            pl.BlockSpec((H, n_out), lambda i: (0, 0)),
            pl.BlockSpec((1, n_out), lambda i: (0, 0)),
        ],
        out_specs=pl.BlockSpec((BLK, n_out), lambda i: (i, 0)),
        out_shape=jax.ShapeDtypeStruct((N_NODES, n_out), jnp.float32),
    )(p, dinv, b2d, wl_t, bl2d)


def _kron_i4(a):
    m, n = a.shape
    eye = jnp.eye(PACK, dtype=a.dtype)
    return jnp.einsum("jk,ab->jakb", eye, a).reshape(PACK * m, PACK * n)


def kernel(price_input, e, concept, Wih, Whh, bih, bhh, W1, b1, W2, b2, Wl, bl):
    x4 = jnp.pad(price_input.reshape(NROWS, XW),
                 ((0, NROWS_PAD - NROWS), (0, 0)))
    w1m_t = W1.reshape(4, H, H).mean(axis=0).T
    wihT = Wih.T
    whhT = Whh.T
    ia = jnp.arange(SEQ * FIN)
    it = jnp.arange(SEQ)
    ii = jnp.arange(FIN)
    E = (ia[None, :, None] == (FIN * it[:, None, None] + ii[None, None, :])
         ).astype(jnp.float32)
    eye = jnp.eye(PACK, dtype=jnp.float32)
    wx = []
    for g in range(3):
        P = jnp.einsum("tai,ib->tab", E, wihT[:, g * H:(g + 1) * H])
        wx.append(jnp.einsum("jk,tab->tjakb", eye, P)
                  .reshape(SEQ, XW, HW))
    wh = [_kron_i4(whhT[:, g * H:(g + 1) * H]) for g in range(3)]
    bg = [jnp.tile((bih[:H] + bhh[:H]).reshape(1, H), (1, PACK)),
          jnp.tile((bih[H:2 * H] + bhh[H:2 * H]).reshape(1, H), (1, PACK)),
          jnp.tile(bih[2 * H:].reshape(1, H), (1, PACK)),
          jnp.tile(bhh[2 * H:].reshape(1, H), (1, PACK))]
    t1 = _tc_gru(x4, wx, wh, bg, _kron_i4(w1m_t))[:NROWS].reshape(N_NODES, H)
    ig = e[0].reshape(IDX_ROWS, CHUNK)
    ie = e[1].reshape(IDX_ROWS, CHUNK)
    cd, cb = _sc_counts(ig, ie)
    dinv, binv = _tc_inv(cd, cb)
    z = jnp.zeros((ROWS_PER_TILE, H), jnp.float32)
    p = _sc_pass(t1, ig, ie, z)
    t2 = _tc_edge_flush(p, binv)
    p = _sc_pass(t2, ie, ig, z)
    t3 = _tc_node_flush(p, dinv, b1.reshape(1, -1), W2.T)
    p = _sc_pass(t3, ig, ie, z)
    t4 = _tc_edge_flush(p, binv)
    p = _sc_pass(t4, ie, ig, z)
    return _tc_final(p, dinv, b2.reshape(1, -1), Wl.T, bl.reshape(1, -1))


# speedup vs baseline: 1.0911x; 1.0911x over previous
import jax
import jax.numpy as jnp
from jax import lax
from jax.experimental import pallas as pl
from jax.experimental.pallas import tpu as pltpu
from jax.experimental.pallas import tpu_sc as plsc

N_NODES = 50000
N_INC = 800000
SEQ = 16
FIN = 6
H = 32
NCORE = 2
NSUB = 16
NTILE = NCORE * NSUB
PER_TILE = N_INC // NTILE
CHUNK = 125
NCHUNK = PER_TILE // CHUNK
IDX_BLK = 40
NREFILL = NCHUNK // IDX_BLK
IDX_ROWS = N_INC // CHUNK
N_PAD = 50048
ROWS_PER_TILE = N_PAD // NSUB
CW = 8
BLK = 2000
GRID = N_NODES // BLK

_SC_MESH = dict(core_axis_name="c", subcore_axis_name="s",
                num_cores=NCORE, num_subcores=NSUB)


NBUF = 4


def _sc_pass_body(table, idxg, idxs, zrows, out, acc, igv, isv,
                  r0, r1, r2, r3, g0, g1, g2, g3, s0, s1, s2, s3):
    c = lax.axis_index("c")
    s = lax.axis_index("s")
    w = c * NSUB + s
    rows = [r0, r1, r2, r3]
    gsem = [g0, g1, g2, g3]
    ssem = [s0, s1, s2, s3]
    sl = pl.ds(s * ROWS_PER_TILE, ROWS_PER_TILE)
    pltpu.sync_copy(zrows, acc.at[sl])
    plsc.subcore_barrier()

    def outer(j, carry):
        base = w * NCHUNK + j * IDX_BLK
        pltpu.sync_copy(idxg.at[pl.ds(base, IDX_BLK)], igv)
        pltpu.sync_copy(idxs.at[pl.ds(base, IDX_BLK)], isv)
        for b in range(NBUF):
            pltpu.async_copy(table.at[igv.at[b]], rows[b], gsem[b])

        def body(k, carry2):
            for b in range(NBUF):
                i = NBUF * k + b
                pltpu.make_async_copy(table.at[igv.at[i]], rows[b],
                                      gsem[b]).wait()
                pltpu.async_copy(rows[b], acc.at[isv.at[i]], ssem[b],
                                 add=True)
            for b in range(NBUF):
                i = NBUF * k + b

                @pl.when(k < IDX_BLK // NBUF - 1)
                def _():
                    pltpu.make_async_copy(rows[b], acc.at[isv.at[i]],
                                          ssem[b]).wait()
                    pltpu.async_copy(table.at[igv.at[i + NBUF]], rows[b],
                                     gsem[b])
            return carry2

        lax.fori_loop(0, IDX_BLK // NBUF, body, 0)
        for b in range(NBUF):
            i = IDX_BLK - NBUF + b
            pltpu.make_async_copy(rows[b], acc.at[isv.at[i]], ssem[b]).wait()
        return carry

    lax.fori_loop(0, NREFILL, outer, 0)
    plsc.subcore_barrier()

    @pl.when(c == 0)
    def _():
        pltpu.sync_copy(acc.at[sl], out.at[0, sl])

    @pl.when(c == 1)
    def _():
        pltpu.sync_copy(acc.at[sl], out.at[1, sl])


def _sc_pass(table, idxg, idxs, zrows):
    return pl.kernel(
        _sc_pass_body,
        out_type=jax.ShapeDtypeStruct((NCORE, N_PAD, H), jnp.float32),
        mesh=plsc.VectorSubcoreMesh(**_SC_MESH),
        scratch_types=[
            pltpu.VMEM_SHARED((N_PAD, H), jnp.float32),
            pltpu.VMEM((IDX_BLK, CHUNK), jnp.int32),
            pltpu.VMEM((IDX_BLK, CHUNK), jnp.int32),
        ] + [pltpu.VMEM((CHUNK, H), jnp.float32)] * NBUF
          + [pltpu.SemaphoreType.DMA] * (2 * NBUF),
        compiler_params=pltpu.CompilerParams(use_tc_tiling_on_sc=False),
    )(table, idxg, idxs, zrows)


def _sc_counts_body(idxg, idxs, zrows, ones, outd, outb, accd, accb,
                    igv, isv, onev):
    c = lax.axis_index("c")
    s = lax.axis_index("s")
    w = c * NSUB + s
    sl = pl.ds(s * ROWS_PER_TILE, ROWS_PER_TILE)
    pltpu.sync_copy(zrows, accd.at[sl])
    pltpu.sync_copy(zrows, accb.at[sl])
    pltpu.sync_copy(ones, onev)
    plsc.subcore_barrier()

    def outer(j, carry):
        base = w * NCHUNK + j * IDX_BLK
        pltpu.sync_copy(idxg.at[pl.ds(base, IDX_BLK)], igv)
        pltpu.sync_copy(idxs.at[pl.ds(base, IDX_BLK)], isv)

        def body(i, carry2):
            pltpu.sync_copy(onev, accd.at[igv.at[i]], add=True)
            pltpu.sync_copy(onev, accb.at[isv.at[i]], add=True)
            return carry2

        lax.fori_loop(0, IDX_BLK, body, 0)
        return carry

    lax.fori_loop(0, NREFILL, outer, 0)
    plsc.subcore_barrier()

    @pl.when(c == 0)
    def _():
        pltpu.sync_copy(accd.at[sl], outd.at[0, sl])
        pltpu.sync_copy(accb.at[sl], outb.at[0, sl])

    @pl.when(c == 1)
    def _():
        pltpu.sync_copy(accd.at[sl], outd.at[1, sl])
        pltpu.sync_copy(accb.at[sl], outb.at[1, sl])


def _sc_counts(idxg, idxs):
    zrows = jnp.zeros((ROWS_PER_TILE, CW), jnp.float32)
    ones = jnp.ones((CHUNK, CW), jnp.float32)
    return pl.kernel(
        _sc_counts_body,
        out_type=(jax.ShapeDtypeStruct((NCORE, N_PAD, CW), jnp.float32),
                  jax.ShapeDtypeStruct((NCORE, N_PAD, CW), jnp.float32)),
        mesh=plsc.VectorSubcoreMesh(**_SC_MESH),
        scratch_types=[
            pltpu.VMEM_SHARED((N_PAD, CW), jnp.float32),
            pltpu.VMEM_SHARED((N_PAD, CW), jnp.float32),
            pltpu.VMEM((IDX_BLK, CHUNK), jnp.int32),
            pltpu.VMEM((IDX_BLK, CHUNK), jnp.int32),
            pltpu.VMEM((CHUNK, CW), jnp.float32),
        ],
        compiler_params=pltpu.CompilerParams(use_tc_tiling_on_sc=False),
    )(idxg, idxs, zrows, ones)


PACK = 4
NROWS = N_NODES // PACK
NROWS_PAD = 12800
GROWS = 3200
GGRID = NROWS_PAD // GROWS
XW = PACK * SEQ * FIN
HW = PACK * H


def _sigm(v):
    return 0.5 * (jnp.tanh(0.5 * v) + 1.0)


def _gru_body(x_ref, wxr_ref, wxz_ref, wxn_ref, whr_ref, whz_ref, whn_ref,
              br_ref, bz_ref, bni_ref, bnh_ref, w1_ref, out_ref):
    x = x_ref[...]
    whr = whr_ref[...]
    whz = whz_ref[...]
    whn = whn_ref[...]
    br = br_ref[...]
    bz = bz_ref[...]
    bni = bni_ref[...]
    bnh = bnh_ref[...]
    h = jnp.zeros((GROWS, HW), jnp.float32)
    for t in range(SEQ):
        gr = (jnp.dot(x, wxr_ref[t], preferred_element_type=jnp.float32)
              + jnp.dot(h, whr, preferred_element_type=jnp.float32) + br)
        gz = (jnp.dot(x, wxz_ref[t], preferred_element_type=jnp.float32)
              + jnp.dot(h, whz, preferred_element_type=jnp.float32) + bz)
        r = _sigm(gr)
        z = _sigm(gz)
        hn = jnp.dot(h, whn, preferred_element_type=jnp.float32) + bnh
        gn = jnp.dot(x, wxn_ref[t], preferred_element_type=jnp.float32) + bni
        n = jnp.tanh(gn + r * hn)
        h = (1.0 - z) * n + z * h
    o = jnp.where(h > 0, h, 0.01 * h)
    out_ref[...] = jnp.dot(o, w1_ref[...], preferred_element_type=jnp.float32)


def _tc_gru(x4, wx, wh, bg, w14):
    full = lambda shape: pl.BlockSpec(shape, lambda i: tuple(0 for _ in shape))
    return pl.pallas_call(
        _gru_body,
        grid=(GGRID,),
        in_specs=[
            pl.BlockSpec((GROWS, XW), lambda i: (i, 0)),
            full((SEQ, XW, HW)),
            full((SEQ, XW, HW)),
            full((SEQ, XW, HW)),
            full((HW, HW)),
            full((HW, HW)),
            full((HW, HW)),
            full((1, HW)),
            full((1, HW)),
            full((1, HW)),
            full((1, HW)),
            full((HW, HW)),
        ],
        out_specs=pl.BlockSpec((GROWS, HW), lambda i: (i, 0)),
        out_shape=jax.ShapeDtypeStruct((NROWS_PAD, HW), jnp.float32),
    )(x4, wx[0], wx[1], wx[2], wh[0], wh[1], wh[2],
      bg[0], bg[1], bg[2], bg[3], w14)


def _inv_body(cd_ref, cb_ref, dinv_ref, binv_ref):
    cd = cd_ref[0, :, :1] + cd_ref[1, :, :1]
    cb = cb_ref[0, :, :1] + cb_ref[1, :, :1]
    dinv_ref[...] = jnp.where(cd > 0, 1.0 / cd, 0.0)
    binv_ref[...] = jnp.where(cb > 0, 1.0 / cb, 0.0)


def _tc_inv(cd, cb):
    return pl.pallas_call(
        _inv_body,
        grid=(GRID,),
        in_specs=[
            pl.BlockSpec((NCORE, BLK, CW), lambda i: (0, i, 0)),
            pl.BlockSpec((NCORE, BLK, CW), lambda i: (0, i, 0)),
        ],
        out_specs=[
            pl.BlockSpec((BLK, 1), lambda i: (i, 0)),
            pl.BlockSpec((BLK, 1), lambda i: (i, 0)),
        ],
        out_shape=[jax.ShapeDtypeStruct((N_NODES, 1), jnp.float32),
                   jax.ShapeDtypeStruct((N_NODES, 1), jnp.float32)],
    )(cd, cb)


def _edge_flush_body(p_ref, binv_ref, out_ref):
    out_ref[...] = (p_ref[0] + p_ref[1]) * binv_ref[...]


def _tc_edge_flush(p, binv):
    return pl.pallas_call(
        _edge_flush_body,
        grid=(GRID,),
        in_specs=[
            pl.BlockSpec((NCORE, BLK, H), lambda i: (0, i, 0)),
            pl.BlockSpec((BLK, 1), lambda i: (i, 0)),
        ],
        out_specs=pl.BlockSpec((BLK, H), lambda i: (i, 0)),
        out_shape=jax.ShapeDtypeStruct((N_NODES, H), jnp.float32),
    )(p, binv)


def _node_flush_body(p_ref, dinv_ref, b_ref, w_ref, out_ref):
    y = (p_ref[0] + p_ref[1]) * dinv_ref[...] + b_ref[...]
    y = jnp.where(y > 0, y, 0.2 * y)
    out_ref[...] = jnp.dot(y, w_ref[...], preferred_element_type=jnp.float32)


def _tc_node_flush(p, dinv, b2d, w_t):
    return pl.pallas_call(
        _node_flush_body,
        grid=(GRID,),
        in_specs=[
            pl.BlockSpec((NCORE, BLK, H), lambda i: (0, i, 0)),
            pl.BlockSpec((BLK, 1), lambda i: (i, 0)),
            pl.BlockSpec((1, H), lambda i: (0, 0)),
            pl.BlockSpec((H, H), lambda i: (0, 0)),
        ],
        out_specs=pl.BlockSpec((BLK, H), lambda i: (i, 0)),
        out_shape=jax.ShapeDtypeStruct((N_NODES, H), jnp.float32),
    )(p, dinv, b2d, w_t)


def _final_body(p_ref, dinv_ref, b_ref, wl_ref, bl_ref, out_ref):
    y = (p_ref[0] + p_ref[1]) * dinv_ref[...] + b_ref[...]
    y = jnp.where(y > 0, y, 0.2 * y)
    o = jnp.dot(y, wl_ref[...], preferred_element_type=jnp.float32) + bl_ref[...]
    out_ref[...] = jnp.where(o > 0, o, 0.01 * o)


def _tc_final(p, dinv, b2d, wl_t, bl2d):
    n_out = wl_t.shape[1]
    return pl.pallas_call(
        _final_body,
        grid=(GRID,),
        in_specs=[
            pl.BlockSpec((NCORE, BLK, H), lambda i: (0, i, 0)),
            pl.BlockSpec((BLK, 1), lambda i: (i, 0)),
            pl.BlockSpec((1, H), lambda i: (0, 0)),
            pl.BlockSpec((H, n_out), lambda i: (0, 0)),
            pl.BlockSpec((1, n_out), lambda i: (0, 0)),
        ],
        out_specs=pl.BlockSpec((BLK, n_out), lambda i: (i, 0)),
        out_shape=jax.ShapeDtypeStruct((N_NODES, n_out), jnp.float32),
    )(p, dinv, b2d, wl_t, bl2d)


def _kron_i4(a):
    m, n = a.shape
    eye = jnp.eye(PACK, dtype=a.dtype)
    return jnp.einsum("jk,ab->jakb", eye, a).reshape(PACK * m, PACK * n)


def kernel(price_input, e, concept, Wih, Whh, bih, bhh, W1, b1, W2, b2, Wl, bl):
    x4 = jnp.pad(price_input.reshape(NROWS, XW),
                 ((0, NROWS_PAD - NROWS), (0, 0)))
    w1m_t = W1.reshape(4, H, H).mean(axis=0).T
    wihT = Wih.T
    whhT = Whh.T
    ia = jnp.arange(SEQ * FIN)
    it = jnp.arange(SEQ)
    ii = jnp.arange(FIN)
    E = (ia[None, :, None] == (FIN * it[:, None, None] + ii[None, None, :])
         ).astype(jnp.float32)
    eye = jnp.eye(PACK, dtype=jnp.float32)
    wx = []
    for g in range(3):
        P = jnp.einsum("tai,ib->tab", E, wihT[:, g * H:(g + 1) * H])
        wx.append(jnp.einsum("jk,tab->tjakb", eye, P)
                  .reshape(SEQ, XW, HW))
    wh = [_kron_i4(whhT[:, g * H:(g + 1) * H]) for g in range(3)]
    bg = [jnp.tile((bih[:H] + bhh[:H]).reshape(1, H), (1, PACK)),
          jnp.tile((bih[H:2 * H] + bhh[H:2 * H]).reshape(1, H), (1, PACK)),
          jnp.tile(bih[2 * H:].reshape(1, H), (1, PACK)),
          jnp.tile(bhh[2 * H:].reshape(1, H), (1, PACK))]
    t1 = _tc_gru(x4, wx, wh, bg, _kron_i4(w1m_t))[:NROWS].reshape(N_NODES, H)
    ig = e[0].reshape(IDX_ROWS, CHUNK)
    ie = e[1].reshape(IDX_ROWS, CHUNK)
    cd, cb = _sc_counts(ig, ie)
    dinv, binv = _tc_inv(cd, cb)
    z = jnp.zeros((ROWS_PER_TILE, H), jnp.float32)
    p = _sc_pass(t1, ig, ie, z)
    t2 = _tc_edge_flush(p, binv)
    p = _sc_pass(t2, ie, ig, z)
    t3 = _tc_node_flush(p, dinv, b1.reshape(1, -1), W2.T)
    p = _sc_pass(t3, ig, ie, z)
    t4 = _tc_edge_flush(p, binv)
    p = _sc_pass(t4, ie, ig, z)
    return _tc_final(p, dinv, b2.reshape(1, -1), Wl.T, bl.reshape(1, -1))


# speedup vs baseline: 1.0915x; 1.0004x over previous
import jax
import jax.numpy as jnp
from jax import lax
from jax.experimental import pallas as pl
from jax.experimental.pallas import tpu as pltpu
from jax.experimental.pallas import tpu_sc as plsc

N_NODES = 50000
N_INC = 800000
SEQ = 16
FIN = 6
H = 32
NCORE = 2
NSUB = 16
NTILE = NCORE * NSUB
PER_TILE = N_INC // NTILE
CHUNK = 125
NCHUNK = PER_TILE // CHUNK
IDX_BLK = 40
NREFILL = NCHUNK // IDX_BLK
IDX_ROWS = N_INC // CHUNK
N_PAD = 50048
ROWS_PER_TILE = N_PAD // NSUB
CW = 8
BLK = 2000
GRID = N_NODES // BLK

_SC_MESH = dict(core_axis_name="c", subcore_axis_name="s",
                num_cores=NCORE, num_subcores=NSUB)


NBUF = 4


def _sc_pass_body(table, idxg, idxs, zrows, out, acc, igv, isv,
                  r0, r1, r2, r3, g0, g1, g2, g3, s0, s1, s2, s3):
    c = lax.axis_index("c")
    s = lax.axis_index("s")
    w = c * NSUB + s
    rows = [r0, r1, r2, r3]
    gsem = [g0, g1, g2, g3]
    ssem = [s0, s1, s2, s3]
    sl = pl.ds(s * ROWS_PER_TILE, ROWS_PER_TILE)
    pltpu.sync_copy(zrows, acc.at[sl])
    plsc.subcore_barrier()

    def outer(j, carry):
        base = w * NCHUNK + j * IDX_BLK
        pltpu.sync_copy(idxg.at[pl.ds(base, IDX_BLK)], igv)
        pltpu.sync_copy(idxs.at[pl.ds(base, IDX_BLK)], isv)
        for b in range(NBUF):
            pltpu.async_copy(table.at[igv.at[b]], rows[b], gsem[b])

        def body(k, carry2):
            for b in range(NBUF):
                i = NBUF * k + b
                pltpu.make_async_copy(table.at[igv.at[i]], rows[b],
                                      gsem[b]).wait()
                pltpu.async_copy(rows[b], acc.at[isv.at[i]], ssem[b],
                                 add=True)
            for b in range(NBUF):
                i = NBUF * k + b

                @pl.when(k < IDX_BLK // NBUF - 1)
                def _():
                    pltpu.make_async_copy(rows[b], acc.at[isv.at[i]],
                                          ssem[b]).wait()
                    pltpu.async_copy(table.at[igv.at[i + NBUF]], rows[b],
                                     gsem[b])
            return carry2

        lax.fori_loop(0, IDX_BLK // NBUF, body, 0)
        for b in range(NBUF):
            i = IDX_BLK - NBUF + b
            pltpu.make_async_copy(rows[b], acc.at[isv.at[i]], ssem[b]).wait()
        return carry

    lax.fori_loop(0, NREFILL, outer, 0)
    plsc.subcore_barrier()

    @pl.when(c == 0)
    def _():
        pltpu.sync_copy(acc.at[sl], out.at[0, sl])

    @pl.when(c == 1)
    def _():
        pltpu.sync_copy(acc.at[sl], out.at[1, sl])


def _sc_pass(table, idxg, idxs, zrows):
    return pl.kernel(
        _sc_pass_body,
        out_type=jax.ShapeDtypeStruct((NCORE, N_PAD, H), jnp.float32),
        mesh=plsc.VectorSubcoreMesh(**_SC_MESH),
        scratch_types=[
            pltpu.VMEM_SHARED((N_PAD, H), jnp.float32),
            pltpu.VMEM((IDX_BLK, CHUNK), jnp.int32),
            pltpu.VMEM((IDX_BLK, CHUNK), jnp.int32),
        ] + [pltpu.VMEM((CHUNK, H), jnp.float32)] * NBUF
          + [pltpu.SemaphoreType.DMA] * (2 * NBUF),
        compiler_params=pltpu.CompilerParams(use_tc_tiling_on_sc=False),
    )(table, idxg, idxs, zrows)


def _sc_counts_body(idxg, idxs, zrows, ones, outd, outb, accd, accb,
                    igv, isv, onev, semd, semb):
    c = lax.axis_index("c")
    s = lax.axis_index("s")
    w = c * NSUB + s
    sl = pl.ds(s * ROWS_PER_TILE, ROWS_PER_TILE)
    pltpu.sync_copy(zrows, accd.at[sl])
    pltpu.sync_copy(zrows, accb.at[sl])
    pltpu.sync_copy(ones, onev)
    plsc.subcore_barrier()

    def outer(j, carry):
        base = w * NCHUNK + j * IDX_BLK
        pltpu.sync_copy(idxg.at[pl.ds(base, IDX_BLK)], igv)
        pltpu.sync_copy(idxs.at[pl.ds(base, IDX_BLK)], isv)

        def body(k, carry2):
            for b in range(NBUF):
                i = NBUF * k + b
                pltpu.async_copy(onev, accd.at[igv.at[i]], semd, add=True)
                pltpu.async_copy(onev, accb.at[isv.at[i]], semb, add=True)
            for b in range(NBUF):
                i = NBUF * k + b
                pltpu.make_async_copy(onev, accd.at[igv.at[i]], semd).wait()
                pltpu.make_async_copy(onev, accb.at[isv.at[i]], semb).wait()
            return carry2

        lax.fori_loop(0, IDX_BLK // NBUF, body, 0)
        return carry

    lax.fori_loop(0, NREFILL, outer, 0)
    plsc.subcore_barrier()

    @pl.when(c == 0)
    def _():
        pltpu.sync_copy(accd.at[sl], outd.at[0, sl])
        pltpu.sync_copy(accb.at[sl], outb.at[0, sl])

    @pl.when(c == 1)
    def _():
        pltpu.sync_copy(accd.at[sl], outd.at[1, sl])
        pltpu.sync_copy(accb.at[sl], outb.at[1, sl])


def _sc_counts(idxg, idxs):
    zrows = jnp.zeros((ROWS_PER_TILE, CW), jnp.float32)
    ones = jnp.ones((CHUNK, CW), jnp.float32)
    return pl.kernel(
        _sc_counts_body,
        out_type=(jax.ShapeDtypeStruct((NCORE, N_PAD, CW), jnp.float32),
                  jax.ShapeDtypeStruct((NCORE, N_PAD, CW), jnp.float32)),
        mesh=plsc.VectorSubcoreMesh(**_SC_MESH),
        scratch_types=[
            pltpu.VMEM_SHARED((N_PAD, CW), jnp.float32),
            pltpu.VMEM_SHARED((N_PAD, CW), jnp.float32),
            pltpu.VMEM((IDX_BLK, CHUNK), jnp.int32),
            pltpu.VMEM((IDX_BLK, CHUNK), jnp.int32),
            pltpu.VMEM((CHUNK, CW), jnp.float32),
            pltpu.SemaphoreType.DMA,
            pltpu.SemaphoreType.DMA,
        ],
        compiler_params=pltpu.CompilerParams(use_tc_tiling_on_sc=False),
    )(idxg, idxs, zrows, ones)


PACK = 4
NROWS = N_NODES // PACK
NROWS_PAD = 12800
GROWS = 3200
GGRID = NROWS_PAD // GROWS
XW = PACK * SEQ * FIN
HW = PACK * H


def _sigm(v):
    return 0.5 * (jnp.tanh(0.5 * v) + 1.0)


def _gru_body(x_ref, wxr_ref, wxz_ref, wxn_ref, whr_ref, whz_ref, whn_ref,
              br_ref, bz_ref, bni_ref, bnh_ref, w1_ref, out_ref):
    x = x_ref[...]
    whr = whr_ref[...]
    whz = whz_ref[...]
    whn = whn_ref[...]
    br = br_ref[...]
    bz = bz_ref[...]
    bni = bni_ref[...]
    bnh = bnh_ref[...]
    h = jnp.zeros((GROWS, HW), jnp.float32)
    for t in range(SEQ):
        gr = (jnp.dot(x, wxr_ref[t], preferred_element_type=jnp.float32)
              + jnp.dot(h, whr, preferred_element_type=jnp.float32) + br)
        gz = (jnp.dot(x, wxz_ref[t], preferred_element_type=jnp.float32)
              + jnp.dot(h, whz, preferred_element_type=jnp.float32) + bz)
        r = _sigm(gr)
        z = _sigm(gz)
        hn = jnp.dot(h, whn, preferred_element_type=jnp.float32) + bnh
        gn = jnp.dot(x, wxn_ref[t], preferred_element_type=jnp.float32) + bni
        n = jnp.tanh(gn + r * hn)
        h = (1.0 - z) * n + z * h
    o = jnp.where(h > 0, h, 0.01 * h)
    out_ref[...] = jnp.dot(o, w1_ref[...], preferred_element_type=jnp.float32)


def _tc_gru(x4, wx, wh, bg, w14):
    full = lambda shape: pl.BlockSpec(shape, lambda i: tuple(0 for _ in shape))
    return pl.pallas_call(
        _gru_body,
        grid=(GGRID,),
        in_specs=[
            pl.BlockSpec((GROWS, XW), lambda i: (i, 0)),
            full((SEQ, XW, HW)),
            full((SEQ, XW, HW)),
            full((SEQ, XW, HW)),
            full((HW, HW)),
            full((HW, HW)),
            full((HW, HW)),
            full((1, HW)),
            full((1, HW)),
            full((1, HW)),
            full((1, HW)),
            full((HW, HW)),
        ],
        out_specs=pl.BlockSpec((GROWS, HW), lambda i: (i, 0)),
        out_shape=jax.ShapeDtypeStruct((NROWS_PAD, HW), jnp.float32),
    )(x4, wx[0], wx[1], wx[2], wh[0], wh[1], wh[2],
      bg[0], bg[1], bg[2], bg[3], w14)


def _inv_body(cd_ref, cb_ref, dinv_ref, binv_ref):
    cd = cd_ref[0, :, :1] + cd_ref[1, :, :1]
    cb = cb_ref[0, :, :1] + cb_ref[1, :, :1]
    dinv_ref[...] = jnp.where(cd > 0, 1.0 / cd, 0.0)
    binv_ref[...] = jnp.where(cb > 0, 1.0 / cb, 0.0)


def _tc_inv(cd, cb):
    return pl.pallas_call(
        _inv_body,
        grid=(GRID,),
        in_specs=[
            pl.BlockSpec((NCORE, BLK, CW), lambda i: (0, i, 0)),
            pl.BlockSpec((NCORE, BLK, CW), lambda i: (0, i, 0)),
        ],
        out_specs=[
            pl.BlockSpec((BLK, 1), lambda i: (i, 0)),
            pl.BlockSpec((BLK, 1), lambda i: (i, 0)),
        ],
        out_shape=[jax.ShapeDtypeStruct((N_NODES, 1), jnp.float32),
                   jax.ShapeDtypeStruct((N_NODES, 1), jnp.float32)],
    )(cd, cb)


def _edge_flush_body(p_ref, binv_ref, out_ref):
    out_ref[...] = (p_ref[0] + p_ref[1]) * binv_ref[...]


def _tc_edge_flush(p, binv):
    return pl.pallas_call(
        _edge_flush_body,
        grid=(GRID,),
        in_specs=[
            pl.BlockSpec((NCORE, BLK, H), lambda i: (0, i, 0)),
            pl.BlockSpec((BLK, 1), lambda i: (i, 0)),
        ],
        out_specs=pl.BlockSpec((BLK, H), lambda i: (i, 0)),
        out_shape=jax.ShapeDtypeStruct((N_NODES, H), jnp.float32),
    )(p, binv)


def _node_flush_body(p_ref, dinv_ref, b_ref, w_ref, out_ref):
    y = (p_ref[0] + p_ref[1]) * dinv_ref[...] + b_ref[...]
    y = jnp.where(y > 0, y, 0.2 * y)
    out_ref[...] = jnp.dot(y, w_ref[...], preferred_element_type=jnp.float32)


def _tc_node_flush(p, dinv, b2d, w_t):
    return pl.pallas_call(
        _node_flush_body,
        grid=(GRID,),
        in_specs=[
            pl.BlockSpec((NCORE, BLK, H), lambda i: (0, i, 0)),
            pl.BlockSpec((BLK, 1), lambda i: (i, 0)),
            pl.BlockSpec((1, H), lambda i: (0, 0)),
            pl.BlockSpec((H, H), lambda i: (0, 0)),
        ],
        out_specs=pl.BlockSpec((BLK, H), lambda i: (i, 0)),
        out_shape=jax.ShapeDtypeStruct((N_NODES, H), jnp.float32),
    )(p, dinv, b2d, w_t)


def _final_body(p_ref, dinv_ref, b_ref, wl_ref, bl_ref, out_ref):
    y = (p_ref[0] + p_ref[1]) * dinv_ref[...] + b_ref[...]
    y = jnp.where(y > 0, y, 0.2 * y)
    o = jnp.dot(y, wl_ref[...], preferred_element_type=jnp.float32) + bl_ref[...]
    out_ref[...] = jnp.where(o > 0, o, 0.01 * o)


def _tc_final(p, dinv, b2d, wl_t, bl2d):
    n_out = wl_t.shape[1]
    return pl.pallas_call(
        _final_body,
        grid=(GRID,),
        in_specs=[
            pl.BlockSpec((NCORE, BLK, H), lambda i: (0, i, 0)),
            pl.BlockSpec((BLK, 1), lambda i: (i, 0)),
            pl.BlockSpec((1, H), lambda i: (0, 0)),
            pl.BlockSpec((H, n_out), lambda i: (0, 0)),
            pl.BlockSpec((1, n_out), lambda i: (0, 0)),
        ],
        out_specs=pl.BlockSpec((BLK, n_out), lambda i: (i, 0)),
        out_shape=jax.ShapeDtypeStruct((N_NODES, n_out), jnp.float32),
    )(p, dinv, b2d, wl_t, bl2d)


def _kron_i4(a):
    m, n = a.shape
    eye = jnp.eye(PACK, dtype=a.dtype)
    return jnp.einsum("jk,ab->jakb", eye, a).reshape(PACK * m, PACK * n)


def kernel(price_input, e, concept, Wih, Whh, bih, bhh, W1, b1, W2, b2, Wl, bl):
    x4 = jnp.pad(price_input.reshape(NROWS, XW),
                 ((0, NROWS_PAD - NROWS), (0, 0)))
    w1m_t = W1.reshape(4, H, H).mean(axis=0).T
    wihT = Wih.T
    whhT = Whh.T
    ia = jnp.arange(SEQ * FIN)
    it = jnp.arange(SEQ)
    ii = jnp.arange(FIN)
    E = (ia[None, :, None] == (FIN * it[:, None, None] + ii[None, None, :])
         ).astype(jnp.float32)
    eye = jnp.eye(PACK, dtype=jnp.float32)
    wx = []
    for g in range(3):
        P = jnp.einsum("tai,ib->tab", E, wihT[:, g * H:(g + 1) * H])
        wx.append(jnp.einsum("jk,tab->tjakb", eye, P)
                  .reshape(SEQ, XW, HW))
    wh = [_kron_i4(whhT[:, g * H:(g + 1) * H]) for g in range(3)]
    bg = [jnp.tile((bih[:H] + bhh[:H]).reshape(1, H), (1, PACK)),
          jnp.tile((bih[H:2 * H] + bhh[H:2 * H]).reshape(1, H), (1, PACK)),
          jnp.tile(bih[2 * H:].reshape(1, H), (1, PACK)),
          jnp.tile(bhh[2 * H:].reshape(1, H), (1, PACK))]
    t1 = _tc_gru(x4, wx, wh, bg, _kron_i4(w1m_t))[:NROWS].reshape(N_NODES, H)
    ig = e[0].reshape(IDX_ROWS, CHUNK)
    ie = e[1].reshape(IDX_ROWS, CHUNK)
    cd, cb = _sc_counts(ig, ie)
    dinv, binv = _tc_inv(cd, cb)
    z = jnp.zeros((ROWS_PER_TILE, H), jnp.float32)
    p = _sc_pass(t1, ig, ie, z)
    t2 = _tc_edge_flush(p, binv)
    p = _sc_pass(t2, ie, ig, z)
    t3 = _tc_node_flush(p, dinv, b1.reshape(1, -1), W2.T)
    p = _sc_pass(t3, ig, ie, z)
    t4 = _tc_edge_flush(p, binv)
    p = _sc_pass(t4, ie, ig, z)
    return _tc_final(p, dinv, b2.reshape(1, -1), Wl.T, bl.reshape(1, -1))


# speedup vs baseline: 1.0943x; 1.0026x over previous
import jax
import jax.numpy as jnp
from jax import lax
from jax.experimental import pallas as pl
from jax.experimental.pallas import tpu as pltpu
from jax.experimental.pallas import tpu_sc as plsc

N_NODES = 50000
N_INC = 800000
SEQ = 16
FIN = 6
H = 32
NCORE = 2
NSUB = 16
NTILE = NCORE * NSUB
PER_TILE = N_INC // NTILE
CHUNK = 125
NCHUNK = PER_TILE // CHUNK
IDX_BLK = 40
NREFILL = NCHUNK // IDX_BLK
IDX_ROWS = N_INC // CHUNK
N_PAD = 50048
ROWS_PER_TILE = N_PAD // NSUB
CW = 8
BLK = 2000
GRID = N_NODES // BLK

_SC_MESH = dict(core_axis_name="c", subcore_axis_name="s",
                num_cores=NCORE, num_subcores=NSUB)


NBUF = 4


def _sc_pass_body(table, idxg, idxs, zrows, out, acc, igv, isv,
                  r0, r1, r2, r3, g0, g1, g2, g3, s0, s1, s2, s3):
    c = lax.axis_index("c")
    s = lax.axis_index("s")
    w = c * NSUB + s
    rows = [r0, r1, r2, r3]
    gsem = [g0, g1, g2, g3]
    ssem = [s0, s1, s2, s3]
    sl = pl.ds(s * ROWS_PER_TILE, ROWS_PER_TILE)
    pltpu.sync_copy(zrows, acc.at[sl])
    plsc.subcore_barrier()

    def outer(j, carry):
        base = w * NCHUNK + j * IDX_BLK
        pltpu.sync_copy(idxg.at[pl.ds(base, IDX_BLK)], igv)
        pltpu.sync_copy(idxs.at[pl.ds(base, IDX_BLK)], isv)
        for b in range(NBUF):
            pltpu.async_copy(table.at[igv.at[b]], rows[b], gsem[b])

        def body(k, carry2):
            for b in range(NBUF):
                i = NBUF * k + b
                pltpu.make_async_copy(table.at[igv.at[i]], rows[b],
                                      gsem[b]).wait()
                pltpu.async_copy(rows[b], acc.at[isv.at[i]], ssem[b],
                                 add=True)
            for b in range(NBUF):
                i = NBUF * k + b

                @pl.when(k < IDX_BLK // NBUF - 1)
                def _():
                    pltpu.make_async_copy(rows[b], acc.at[isv.at[i]],
                                          ssem[b]).wait()
                    pltpu.async_copy(table.at[igv.at[i + NBUF]], rows[b],
                                     gsem[b])
            return carry2

        lax.fori_loop(0, IDX_BLK // NBUF, body, 0)
        for b in range(NBUF):
            i = IDX_BLK - NBUF + b
            pltpu.make_async_copy(rows[b], acc.at[isv.at[i]], ssem[b]).wait()
        return carry

    lax.fori_loop(0, NREFILL, outer, 0)
    plsc.subcore_barrier()

    @pl.when(c == 0)
    def _():
        pltpu.sync_copy(acc.at[sl], out.at[0, sl])

    @pl.when(c == 1)
    def _():
        pltpu.sync_copy(acc.at[sl], out.at[1, sl])


def _sc_pass(table, idxg, idxs, zrows):
    return pl.kernel(
        _sc_pass_body,
        out_type=jax.ShapeDtypeStruct((NCORE, N_PAD, H), jnp.float32),
        mesh=plsc.VectorSubcoreMesh(**_SC_MESH),
        scratch_types=[
            pltpu.VMEM_SHARED((N_PAD, H), jnp.float32),
            pltpu.VMEM((IDX_BLK, CHUNK), jnp.int32),
            pltpu.VMEM((IDX_BLK, CHUNK), jnp.int32),
        ] + [pltpu.VMEM((CHUNK, H), jnp.float32)] * NBUF
          + [pltpu.SemaphoreType.DMA] * (2 * NBUF),
        compiler_params=pltpu.CompilerParams(use_tc_tiling_on_sc=False),
    )(table, idxg, idxs, zrows)


def _sc_counts_body(idxg, idxs, zrows, ones, outd, outb, accd, accb,
                    igv, isv, onev, semd, semb):
    c = lax.axis_index("c")
    s = lax.axis_index("s")
    w = c * NSUB + s
    sl = pl.ds(s * ROWS_PER_TILE, ROWS_PER_TILE)
    pltpu.sync_copy(zrows, accd.at[sl])
    pltpu.sync_copy(zrows, accb.at[sl])
    pltpu.sync_copy(ones, onev)
    plsc.subcore_barrier()

    def outer(j, carry):
        base = w * NCHUNK + j * IDX_BLK
        pltpu.sync_copy(idxg.at[pl.ds(base, IDX_BLK)], igv)
        pltpu.sync_copy(idxs.at[pl.ds(base, IDX_BLK)], isv)

        def body(k, carry2):
            for b in range(NBUF):
                i = NBUF * k + b
                pltpu.async_copy(onev, accd.at[igv.at[i]], semd, add=True)
                pltpu.async_copy(onev, accb.at[isv.at[i]], semb, add=True)
            for b in range(NBUF):
                i = NBUF * k + b
                pltpu.make_async_copy(onev, accd.at[igv.at[i]], semd).wait()
                pltpu.make_async_copy(onev, accb.at[isv.at[i]], semb).wait()
            return carry2

        lax.fori_loop(0, IDX_BLK // NBUF, body, 0)
        return carry

    lax.fori_loop(0, NREFILL, outer, 0)
    plsc.subcore_barrier()

    @pl.when(c == 0)
    def _():
        pltpu.sync_copy(accd.at[sl], outd.at[0, sl])
        pltpu.sync_copy(accb.at[sl], outb.at[0, sl])

    @pl.when(c == 1)
    def _():
        pltpu.sync_copy(accd.at[sl], outd.at[1, sl])
        pltpu.sync_copy(accb.at[sl], outb.at[1, sl])


def _sc_counts(idxg, idxs):
    zrows = jnp.zeros((ROWS_PER_TILE, CW), jnp.float32)
    ones = jnp.ones((CHUNK, CW), jnp.float32)
    return pl.kernel(
        _sc_counts_body,
        out_type=(jax.ShapeDtypeStruct((NCORE, N_PAD, CW), jnp.float32),
                  jax.ShapeDtypeStruct((NCORE, N_PAD, CW), jnp.float32)),
        mesh=plsc.VectorSubcoreMesh(**_SC_MESH),
        scratch_types=[
            pltpu.VMEM_SHARED((N_PAD, CW), jnp.float32),
            pltpu.VMEM_SHARED((N_PAD, CW), jnp.float32),
            pltpu.VMEM((IDX_BLK, CHUNK), jnp.int32),
            pltpu.VMEM((IDX_BLK, CHUNK), jnp.int32),
            pltpu.VMEM((CHUNK, CW), jnp.float32),
            pltpu.SemaphoreType.DMA,
            pltpu.SemaphoreType.DMA,
        ],
        compiler_params=pltpu.CompilerParams(use_tc_tiling_on_sc=False),
    )(idxg, idxs, zrows, ones)


PACK = 4
NROWS = N_NODES // PACK
NROWS_PAD = 12800
GROWS = 3200
GGRID = NROWS_PAD // GROWS
XW = PACK * SEQ * FIN
HW = PACK * H


def _sigm(v):
    return 0.5 * (jnp.tanh(0.5 * v) + 1.0)


def _gru_body(x_ref, wxr_ref, wxz_ref, wxn_ref, whr_ref, whz_ref, whn_ref,
              br_ref, bz_ref, bni_ref, bnh_ref, w1_ref, out_ref):
    x = x_ref[...]
    whr = whr_ref[...]
    whz = whz_ref[...]
    whn = whn_ref[...]
    br = br_ref[...]
    bz = bz_ref[...]
    bni = bni_ref[...]
    bnh = bnh_ref[...]
    xb = x.astype(jnp.bfloat16)
    h = jnp.zeros((GROWS, HW), jnp.float32)
    for t in range(SEQ):
        gr = (jnp.dot(xb, wxr_ref[t], preferred_element_type=jnp.float32)
              + jnp.dot(h, whr, preferred_element_type=jnp.float32) + br)
        gz = (jnp.dot(xb, wxz_ref[t], preferred_element_type=jnp.float32)
              + jnp.dot(h, whz, preferred_element_type=jnp.float32) + bz)
        r = _sigm(gr)
        z = _sigm(gz)
        hn = jnp.dot(h, whn, preferred_element_type=jnp.float32) + bnh
        gn = jnp.dot(xb, wxn_ref[t], preferred_element_type=jnp.float32) + bni
        n = jnp.tanh(gn + r * hn)
        h = (1.0 - z) * n + z * h
    o = jnp.where(h > 0, h, 0.01 * h)
    out_ref[...] = jnp.dot(o, w1_ref[...], preferred_element_type=jnp.float32)


def _tc_gru(x4, wx, wh, bg, w14):
    full = lambda shape: pl.BlockSpec(shape, lambda i: tuple(0 for _ in shape))
    return pl.pallas_call(
        _gru_body,
        grid=(GGRID,),
        in_specs=[
            pl.BlockSpec((GROWS, XW), lambda i: (i, 0)),
            full((SEQ, XW, HW)),
            full((SEQ, XW, HW)),
            full((SEQ, XW, HW)),
            full((HW, HW)),
            full((HW, HW)),
            full((HW, HW)),
            full((1, HW)),
            full((1, HW)),
            full((1, HW)),
            full((1, HW)),
            full((HW, HW)),
        ],
        out_specs=pl.BlockSpec((GROWS, HW), lambda i: (i, 0)),
        out_shape=jax.ShapeDtypeStruct((NROWS_PAD, HW), jnp.float32),
    )(x4, wx[0], wx[1], wx[2], wh[0], wh[1], wh[2],
      bg[0], bg[1], bg[2], bg[3], w14)


def _inv_body(cd_ref, cb_ref, dinv_ref, binv_ref):
    cd = cd_ref[0, :, :1] + cd_ref[1, :, :1]
    cb = cb_ref[0, :, :1] + cb_ref[1, :, :1]
    dinv_ref[...] = jnp.where(cd > 0, 1.0 / cd, 0.0)
    binv_ref[...] = jnp.where(cb > 0, 1.0 / cb, 0.0)


def _tc_inv(cd, cb):
    return pl.pallas_call(
        _inv_body,
        grid=(GRID,),
        in_specs=[
            pl.BlockSpec((NCORE, BLK, CW), lambda i: (0, i, 0)),
            pl.BlockSpec((NCORE, BLK, CW), lambda i: (0, i, 0)),
        ],
        out_specs=[
            pl.BlockSpec((BLK, 1), lambda i: (i, 0)),
            pl.BlockSpec((BLK, 1), lambda i: (i, 0)),
        ],
        out_shape=[jax.ShapeDtypeStruct((N_NODES, 1), jnp.float32),
                   jax.ShapeDtypeStruct((N_NODES, 1), jnp.float32)],
    )(cd, cb)


def _edge_flush_body(p_ref, binv_ref, out_ref):
    out_ref[...] = (p_ref[0] + p_ref[1]) * binv_ref[...]


def _tc_edge_flush(p, binv):
    return pl.pallas_call(
        _edge_flush_body,
        grid=(GRID,),
        in_specs=[
            pl.BlockSpec((NCORE, BLK, H), lambda i: (0, i, 0)),
            pl.BlockSpec((BLK, 1), lambda i: (i, 0)),
        ],
        out_specs=pl.BlockSpec((BLK, H), lambda i: (i, 0)),
        out_shape=jax.ShapeDtypeStruct((N_NODES, H), jnp.float32),
    )(p, binv)


def _node_flush_body(p_ref, dinv_ref, b_ref, w_ref, out_ref):
    y = (p_ref[0] + p_ref[1]) * dinv_ref[...] + b_ref[...]
    y = jnp.where(y > 0, y, 0.2 * y)
    out_ref[...] = jnp.dot(y, w_ref[...], preferred_element_type=jnp.float32)


def _tc_node_flush(p, dinv, b2d, w_t):
    return pl.pallas_call(
        _node_flush_body,
        grid=(GRID,),
        in_specs=[
            pl.BlockSpec((NCORE, BLK, H), lambda i: (0, i, 0)),
            pl.BlockSpec((BLK, 1), lambda i: (i, 0)),
            pl.BlockSpec((1, H), lambda i: (0, 0)),
            pl.BlockSpec((H, H), lambda i: (0, 0)),
        ],
        out_specs=pl.BlockSpec((BLK, H), lambda i: (i, 0)),
        out_shape=jax.ShapeDtypeStruct((N_NODES, H), jnp.float32),
    )(p, dinv, b2d, w_t)


def _final_body(p_ref, dinv_ref, b_ref, wl_ref, bl_ref, out_ref):
    y = (p_ref[0] + p_ref[1]) * dinv_ref[...] + b_ref[...]
    y = jnp.where(y > 0, y, 0.2 * y)
    o = jnp.dot(y, wl_ref[...], preferred_element_type=jnp.float32) + bl_ref[...]
    out_ref[...] = jnp.where(o > 0, o, 0.01 * o)


def _tc_final(p, dinv, b2d, wl_t, bl2d):
    n_out = wl_t.shape[1]
    return pl.pallas_call(
        _final_body,
        grid=(GRID,),
        in_specs=[
            pl.BlockSpec((NCORE, BLK, H), lambda i: (0, i, 0)),
            pl.BlockSpec((BLK, 1), lambda i: (i, 0)),
            pl.BlockSpec((1, H), lambda i: (0, 0)),
            pl.BlockSpec((H, n_out), lambda i: (0, 0)),
            pl.BlockSpec((1, n_out), lambda i: (0, 0)),
        ],
        out_specs=pl.BlockSpec((BLK, n_out), lambda i: (i, 0)),
        out_shape=jax.ShapeDtypeStruct((N_NODES, n_out), jnp.float32),
    )(p, dinv, b2d, wl_t, bl2d)


def _kron_i4(a):
    m, n = a.shape
    eye = jnp.eye(PACK, dtype=a.dtype)
    return jnp.einsum("jk,ab->jakb", eye, a).reshape(PACK * m, PACK * n)


def kernel(price_input, e, concept, Wih, Whh, bih, bhh, W1, b1, W2, b2, Wl, bl):
    x4 = jnp.pad(price_input.reshape(NROWS, XW),
                 ((0, NROWS_PAD - NROWS), (0, 0)))
    w1m_t = W1.reshape(4, H, H).mean(axis=0).T
    wihT = Wih.T
    whhT = Whh.T
    ia = jnp.arange(SEQ * FIN)
    it = jnp.arange(SEQ)
    ii = jnp.arange(FIN)
    E = (ia[None, :, None] == (FIN * it[:, None, None] + ii[None, None, :])
         ).astype(jnp.float32)
    eye = jnp.eye(PACK, dtype=jnp.float32)
    wx = []
    for g in range(3):
        P = jnp.einsum("tai,ib->tab", E, wihT[:, g * H:(g + 1) * H])
        wx.append(jnp.einsum("jk,tab->tjakb", eye, P)
                  .reshape(SEQ, XW, HW).astype(jnp.bfloat16))
    wh = [_kron_i4(whhT[:, g * H:(g + 1) * H]) for g in range(3)]
    bg = [jnp.tile((bih[:H] + bhh[:H]).reshape(1, H), (1, PACK)),
          jnp.tile((bih[H:2 * H] + bhh[H:2 * H]).reshape(1, H), (1, PACK)),
          jnp.tile(bih[2 * H:].reshape(1, H), (1, PACK)),
          jnp.tile(bhh[2 * H:].reshape(1, H), (1, PACK))]
    t1 = _tc_gru(x4, wx, wh, bg, _kron_i4(w1m_t))[:NROWS].reshape(N_NODES, H)
    ig = e[0].reshape(IDX_ROWS, CHUNK)
    ie = e[1].reshape(IDX_ROWS, CHUNK)
    cd, cb = _sc_counts(ig, ie)
    dinv, binv = _tc_inv(cd, cb)
    z = jnp.zeros((ROWS_PER_TILE, H), jnp.float32)
    p = _sc_pass(t1, ig, ie, z)
    t2 = _tc_edge_flush(p, binv)
    p = _sc_pass(t2, ie, ig, z)
    t3 = _tc_node_flush(p, dinv, b1.reshape(1, -1), W2.T)
    p = _sc_pass(t3, ig, ie, z)
    t4 = _tc_edge_flush(p, binv)
    p = _sc_pass(t4, ie, ig, z)
    return _tc_final(p, dinv, b2.reshape(1, -1), Wl.T, bl.reshape(1, -1))


# speedup vs baseline: 1.1154x; 1.0193x over previous
import jax
import jax.numpy as jnp
from jax import lax
from jax.experimental import pallas as pl
from jax.experimental.pallas import tpu as pltpu
from jax.experimental.pallas import tpu_sc as plsc

N_NODES = 50000
N_INC = 800000
SEQ = 16
FIN = 6
H = 32
NCORE = 2
NSUB = 16
NTILE = NCORE * NSUB
PER_TILE = N_INC // NTILE
CHUNK = 125
NCHUNK = PER_TILE // CHUNK
IDX_BLK = 40
NREFILL = NCHUNK // IDX_BLK
IDX_ROWS = N_INC // CHUNK
N_PAD = 50048
ROWS_PER_TILE = N_PAD // NSUB
CW = 8
BLK = 2000
GRID = N_NODES // BLK

_SC_MESH = dict(core_axis_name="c", subcore_axis_name="s",
                num_cores=NCORE, num_subcores=NSUB)


NBUF = 4


def _sc_pass_body(table, idxg, idxs, zrows, out, acc, igv, isv,
                  r0, r1, r2, r3, g0, g1, g2, g3, s0, s1, s2, s3):
    c = lax.axis_index("c")
    s = lax.axis_index("s")
    w = c * NSUB + s
    rows = [r0, r1, r2, r3]
    gsem = [g0, g1, g2, g3]
    ssem = [s0, s1, s2, s3]
    sl = pl.ds(s * ROWS_PER_TILE, ROWS_PER_TILE)
    pltpu.sync_copy(zrows, acc.at[sl])
    plsc.subcore_barrier()

    def outer(j, carry):
        base = w * NCHUNK + j * IDX_BLK
        pltpu.sync_copy(idxg.at[pl.ds(base, IDX_BLK)], igv)
        pltpu.sync_copy(idxs.at[pl.ds(base, IDX_BLK)], isv)
        for b in range(NBUF):
            pltpu.async_copy(table.at[igv.at[b]], rows[b], gsem[b])

        def body(k, carry2):
            for b in range(NBUF):
                i = NBUF * k + b
                pltpu.make_async_copy(table.at[igv.at[i]], rows[b],
                                      gsem[b]).wait()
                pltpu.async_copy(rows[b], acc.at[isv.at[i]], ssem[b],
                                 add=True)
            for b in range(NBUF):
                i = NBUF * k + b

                @pl.when(k < IDX_BLK // NBUF - 1)
                def _():
                    pltpu.make_async_copy(rows[b], acc.at[isv.at[i]],
                                          ssem[b]).wait()
                    pltpu.async_copy(table.at[igv.at[i + NBUF]], rows[b],
                                     gsem[b])
            return carry2

        lax.fori_loop(0, IDX_BLK // NBUF, body, 0)
        for b in range(NBUF):
            i = IDX_BLK - NBUF + b
            pltpu.make_async_copy(rows[b], acc.at[isv.at[i]], ssem[b]).wait()
        return carry

    lax.fori_loop(0, NREFILL, outer, 0)
    plsc.subcore_barrier()

    @pl.when(c == 0)
    def _():
        pltpu.sync_copy(acc.at[sl], out.at[0, sl])

    @pl.when(c == 1)
    def _():
        pltpu.sync_copy(acc.at[sl], out.at[1, sl])


def _sc_pass(table, idxg, idxs, zrows):
    return pl.kernel(
        _sc_pass_body,
        out_type=jax.ShapeDtypeStruct((NCORE, N_PAD, H), jnp.float32),
        mesh=plsc.VectorSubcoreMesh(**_SC_MESH),
        scratch_types=[
            pltpu.VMEM_SHARED((N_PAD, H), jnp.float32),
            pltpu.VMEM((IDX_BLK, CHUNK), jnp.int32),
            pltpu.VMEM((IDX_BLK, CHUNK), jnp.int32),
        ] + [pltpu.VMEM((CHUNK, H), jnp.float32)] * NBUF
          + [pltpu.SemaphoreType.DMA] * (2 * NBUF),
        compiler_params=pltpu.CompilerParams(use_tc_tiling_on_sc=False),
    )(table, idxg, idxs, zrows)


def _sc_counts_body(idxg, idxs, zrows, ones, outd, outb, accd, accb,
                    igv, isv, onev, semd, semb):
    c = lax.axis_index("c")
    s = lax.axis_index("s")
    w = c * NSUB + s
    sl = pl.ds(s * ROWS_PER_TILE, ROWS_PER_TILE)
    pltpu.sync_copy(zrows, accd.at[sl])
    pltpu.sync_copy(zrows, accb.at[sl])
    pltpu.sync_copy(ones, onev)
    plsc.subcore_barrier()

    def outer(j, carry):
        base = w * NCHUNK + j * IDX_BLK
        pltpu.sync_copy(idxg.at[pl.ds(base, IDX_BLK)], igv)
        pltpu.sync_copy(idxs.at[pl.ds(base, IDX_BLK)], isv)

        def body(k, carry2):
            for b in range(NBUF):
                i = NBUF * k + b
                pltpu.async_copy(onev, accd.at[igv.at[i]], semd, add=True)
                pltpu.async_copy(onev, accb.at[isv.at[i]], semb, add=True)
            for b in range(NBUF):
                i = NBUF * k + b
                pltpu.make_async_copy(onev, accd.at[igv.at[i]], semd).wait()
                pltpu.make_async_copy(onev, accb.at[isv.at[i]], semb).wait()
            return carry2

        lax.fori_loop(0, IDX_BLK // NBUF, body, 0)
        return carry

    lax.fori_loop(0, NREFILL, outer, 0)
    plsc.subcore_barrier()

    @pl.when(c == 0)
    def _():
        pltpu.sync_copy(accd.at[sl], outd.at[0, sl])
        pltpu.sync_copy(accb.at[sl], outb.at[0, sl])

    @pl.when(c == 1)
    def _():
        pltpu.sync_copy(accd.at[sl], outd.at[1, sl])
        pltpu.sync_copy(accb.at[sl], outb.at[1, sl])


def _sc_counts(idxg, idxs):
    zrows = jnp.zeros((ROWS_PER_TILE, CW), jnp.float32)
    ones = jnp.ones((CHUNK, CW), jnp.float32)
    return pl.kernel(
        _sc_counts_body,
        out_type=(jax.ShapeDtypeStruct((NCORE, N_PAD, CW), jnp.float32),
                  jax.ShapeDtypeStruct((NCORE, N_PAD, CW), jnp.float32)),
        mesh=plsc.VectorSubcoreMesh(**_SC_MESH),
        scratch_types=[
            pltpu.VMEM_SHARED((N_PAD, CW), jnp.float32),
            pltpu.VMEM_SHARED((N_PAD, CW), jnp.float32),
            pltpu.VMEM((IDX_BLK, CHUNK), jnp.int32),
            pltpu.VMEM((IDX_BLK, CHUNK), jnp.int32),
            pltpu.VMEM((CHUNK, CW), jnp.float32),
            pltpu.SemaphoreType.DMA,
            pltpu.SemaphoreType.DMA,
        ],
        compiler_params=pltpu.CompilerParams(use_tc_tiling_on_sc=False),
    )(idxg, idxs, zrows, ones)


PACK = 4
NROWS = N_NODES // PACK
NROWS_PAD = 12800
GROWS = 3200
GGRID = NROWS_PAD // GROWS
XW = PACK * SEQ * FIN
HW = PACK * H


def _sigm(v):
    return 0.5 * (jnp.tanh(0.5 * v) + 1.0)


def _gru_body(x_ref, wxr_ref, wxz_ref, wxn_ref, whr_ref, whz_ref, whn_ref,
              br_ref, bz_ref, bni_ref, bnh_ref, w1_ref, out_ref):
    x = x_ref[...]
    whr = whr_ref[...]
    whz = whz_ref[...]
    whn = whn_ref[...]
    br = br_ref[...]
    bz = bz_ref[...]
    bni = bni_ref[...]
    bnh = bnh_ref[...]
    xb = x
    h = jnp.zeros((GROWS, HW), jnp.float32)
    for t in range(SEQ):
        gr = (jnp.dot(xb, wxr_ref[t], preferred_element_type=jnp.float32)
              + jnp.dot(h, whr, preferred_element_type=jnp.float32) + br)
        gz = (jnp.dot(xb, wxz_ref[t], preferred_element_type=jnp.float32)
              + jnp.dot(h, whz, preferred_element_type=jnp.float32) + bz)
        r = _sigm(gr)
        z = _sigm(gz)
        hn = jnp.dot(h, whn, preferred_element_type=jnp.float32) + bnh
        gn = jnp.dot(xb, wxn_ref[t], preferred_element_type=jnp.float32) + bni
        n = jnp.tanh(gn + r * hn)
        h = (1.0 - z) * n + z * h
    o = jnp.where(h > 0, h, 0.01 * h)
    out_ref[...] = jnp.dot(o, w1_ref[...], preferred_element_type=jnp.float32)


def _tc_gru(x4, wx, wh, bg, w14):
    full = lambda shape: pl.BlockSpec(shape, lambda i: tuple(0 for _ in shape))
    return pl.pallas_call(
        _gru_body,
        grid=(GGRID,),
        in_specs=[
            pl.BlockSpec((GROWS, XW), lambda i: (i, 0)),
            full((SEQ, XW, HW)),
            full((SEQ, XW, HW)),
            full((SEQ, XW, HW)),
            full((HW, HW)),
            full((HW, HW)),
            full((HW, HW)),
            full((1, HW)),
            full((1, HW)),
            full((1, HW)),
            full((1, HW)),
            full((HW, HW)),
        ],
        out_specs=pl.BlockSpec((GROWS, HW), lambda i: (i, 0)),
        out_shape=jax.ShapeDtypeStruct((NROWS_PAD, HW), jnp.float32),
    )(x4, wx[0], wx[1], wx[2], wh[0], wh[1], wh[2],
      bg[0], bg[1], bg[2], bg[3], w14)


def _inv_body(cd_ref, cb_ref, dinv_ref, binv_ref):
    cd = cd_ref[0, :, :1] + cd_ref[1, :, :1]
    cb = cb_ref[0, :, :1] + cb_ref[1, :, :1]
    dinv_ref[...] = jnp.where(cd > 0, 1.0 / cd, 0.0)
    binv_ref[...] = jnp.where(cb > 0, 1.0 / cb, 0.0)


def _tc_inv(cd, cb):
    return pl.pallas_call(
        _inv_body,
        grid=(GRID,),
        in_specs=[
            pl.BlockSpec((NCORE, BLK, CW), lambda i: (0, i, 0)),
            pl.BlockSpec((NCORE, BLK, CW), lambda i: (0, i, 0)),
        ],
        out_specs=[
            pl.BlockSpec((BLK, 1), lambda i: (i, 0)),
            pl.BlockSpec((BLK, 1), lambda i: (i, 0)),
        ],
        out_shape=[jax.ShapeDtypeStruct((N_NODES, 1), jnp.float32),
                   jax.ShapeDtypeStruct((N_NODES, 1), jnp.float32)],
    )(cd, cb)


def _edge_flush_body(p_ref, binv_ref, out_ref):
    out_ref[...] = (p_ref[0] + p_ref[1]) * binv_ref[...]


def _tc_edge_flush(p, binv):
    return pl.pallas_call(
        _edge_flush_body,
        grid=(GRID,),
        in_specs=[
            pl.BlockSpec((NCORE, BLK, H), lambda i: (0, i, 0)),
            pl.BlockSpec((BLK, 1), lambda i: (i, 0)),
        ],
        out_specs=pl.BlockSpec((BLK, H), lambda i: (i, 0)),
        out_shape=jax.ShapeDtypeStruct((N_NODES, H), jnp.float32),
    )(p, binv)


def _node_flush_body(p_ref, dinv_ref, b_ref, w_ref, out_ref):
    y = (p_ref[0] + p_ref[1]) * dinv_ref[...] + b_ref[...]
    y = jnp.where(y > 0, y, 0.2 * y)
    out_ref[...] = jnp.dot(y, w_ref[...], preferred_element_type=jnp.float32)


def _tc_node_flush(p, dinv, b2d, w_t):
    return pl.pallas_call(
        _node_flush_body,
        grid=(GRID,),
        in_specs=[
            pl.BlockSpec((NCORE, BLK, H), lambda i: (0, i, 0)),
            pl.BlockSpec((BLK, 1), lambda i: (i, 0)),
            pl.BlockSpec((1, H), lambda i: (0, 0)),
            pl.BlockSpec((H, H), lambda i: (0, 0)),
        ],
        out_specs=pl.BlockSpec((BLK, H), lambda i: (i, 0)),
        out_shape=jax.ShapeDtypeStruct((N_NODES, H), jnp.float32),
    )(p, dinv, b2d, w_t)


def _final_body(p_ref, dinv_ref, b_ref, wl_ref, bl_ref, out_ref):
    y = (p_ref[0] + p_ref[1]) * dinv_ref[...] + b_ref[...]
    y = jnp.where(y > 0, y, 0.2 * y)
    o = jnp.dot(y, wl_ref[...], preferred_element_type=jnp.float32) + bl_ref[...]
    out_ref[...] = jnp.where(o > 0, o, 0.01 * o)


def _tc_final(p, dinv, b2d, wl_t, bl2d):
    n_out = wl_t.shape[1]
    return pl.pallas_call(
        _final_body,
        grid=(GRID,),
        in_specs=[
            pl.BlockSpec((NCORE, BLK, H), lambda i: (0, i, 0)),
            pl.BlockSpec((BLK, 1), lambda i: (i, 0)),
            pl.BlockSpec((1, H), lambda i: (0, 0)),
            pl.BlockSpec((H, n_out), lambda i: (0, 0)),
            pl.BlockSpec((1, n_out), lambda i: (0, 0)),
        ],
        out_specs=pl.BlockSpec((BLK, n_out), lambda i: (i, 0)),
        out_shape=jax.ShapeDtypeStruct((N_NODES, n_out), jnp.float32),
    )(p, dinv, b2d, wl_t, bl2d)


def _kron_i4(a):
    m, n = a.shape
    eye = jnp.eye(PACK, dtype=a.dtype)
    return jnp.einsum("jk,ab->jakb", eye, a).reshape(PACK * m, PACK * n)


def kernel(price_input, e, concept, Wih, Whh, bih, bhh, W1, b1, W2, b2, Wl, bl):
    x4 = jnp.pad(price_input.reshape(NROWS, XW),
                 ((0, NROWS_PAD - NROWS), (0, 0))).astype(jnp.bfloat16)
    w1m_t = W1.reshape(4, H, H).mean(axis=0).T
    wihT = Wih.T
    whhT = Whh.T
    ia = jnp.arange(SEQ * FIN)
    it = jnp.arange(SEQ)
    ii = jnp.arange(FIN)
    E = (ia[None, :, None] == (FIN * it[:, None, None] + ii[None, None, :])
         ).astype(jnp.float32)
    eye = jnp.eye(PACK, dtype=jnp.float32)
    wx = []
    for g in range(3):
        P = jnp.einsum("tai,ib->tab", E, wihT[:, g * H:(g + 1) * H])
        wx.append(jnp.einsum("jk,tab->tjakb", eye, P)
                  .reshape(SEQ, XW, HW).astype(jnp.bfloat16))
    wh = [_kron_i4(whhT[:, g * H:(g + 1) * H]) for g in range(3)]
    bg = [jnp.tile((bih[:H] + bhh[:H]).reshape(1, H), (1, PACK)),
          jnp.tile((bih[H:2 * H] + bhh[H:2 * H]).reshape(1, H), (1, PACK)),
          jnp.tile(bih[2 * H:].reshape(1, H), (1, PACK)),
          jnp.tile(bhh[2 * H:].reshape(1, H), (1, PACK))]
    t1 = _tc_gru(x4, wx, wh, bg, _kron_i4(w1m_t))[:NROWS].reshape(N_NODES, H)
    ig = e[0].reshape(IDX_ROWS, CHUNK)
    ie = e[1].reshape(IDX_ROWS, CHUNK)
    cd, cb = _sc_counts(ig, ie)
    dinv, binv = _tc_inv(cd, cb)
    z = jnp.zeros((ROWS_PER_TILE, H), jnp.float32)
    p = _sc_pass(t1, ig, ie, z)
    t2 = _tc_edge_flush(p, binv)
    p = _sc_pass(t2, ie, ig, z)
    t3 = _tc_node_flush(p, dinv, b1.reshape(1, -1), W2.T)
    p = _sc_pass(t3, ig, ie, z)
    t4 = _tc_edge_flush(p, binv)
    p = _sc_pass(t4, ie, ig, z)
    return _tc_final(p, dinv, b2.reshape(1, -1), Wl.T, bl.reshape(1, -1))
